# baseline (speedup 1.0000x reference)
import functools

import jax
import jax.numpy as jnp
from jax import lax
from jax.experimental import pallas as pl
from jax.experimental.pallas import tpu as pltpu
from jax.experimental.pallas import tpu_sc as plsc

NUM_SEGMENTS = 4096
T = 131072
D = 64
DW = D // 2
NC = 2
NS = 16
NW = NC * NS
PER_W = T // NW
CHUNK = 128
NCHUNK = PER_W // CHUNK
ACCW = 32

NTYPES = 50000
NRELS = 1000
SLAB = 512
TPAD = ((NTYPES + SLAB - 1) // SLAB) * SLAB
RPAD = ((NRELS + SLAB - 1) // SLAB) * SLAB
NTCHUNK = TPAD // SLAB


def _sc_kernel(dom2d, ran2d, rel2d, seg2d, tembT, rembT, zeros_hbm):
    mesh = plsc.VectorSubcoreMesh(core_axis_name="c", subcore_axis_name="s")

    @functools.partial(
        pl.kernel,
        out_type=(
            jax.ShapeDtypeStruct((NC, NUM_SEGMENTS, ACCW), jnp.float32),
            jax.ShapeDtypeStruct((NC * TPAD, DW), jnp.int32),
            jax.ShapeDtypeStruct((NC * RPAD, DW), jnp.int32),
        ),
        mesh=mesh,
        scratch_types=[
            pltpu.VMEM((NCHUNK, CHUNK), jnp.int32),
            pltpu.VMEM((NCHUNK, CHUNK), jnp.int32),
            pltpu.VMEM((NCHUNK, CHUNK), jnp.int32),
            pltpu.VMEM((NCHUNK, CHUNK), jnp.int32),
            pltpu.VMEM((64, SLAB), jnp.float32),
            pltpu.VMEM((SLAB, DW), jnp.int32),
            pltpu.VMEM((2, CHUNK, DW), jnp.int32),
            pltpu.VMEM((2, CHUNK, DW), jnp.int32),
            pltpu.VMEM((2, CHUNK, DW), jnp.int32),
            pltpu.VMEM((2, CHUNK, ACCW), jnp.float32),
            pltpu.VMEM_SHARED((NUM_SEGMENTS, ACCW), jnp.float32),
            pltpu.SemaphoreType.DMA,
            pltpu.SemaphoreType.DMA,
            pltpu.SemaphoreType.DMA,
        ],
        compiler_params=pltpu.CompilerParams(use_tc_tiling_on_sc=False,
                                             needs_layout_passes=False),
    )
    def k(dom_h, ran_h, rel_h, seg_h, tembT_h, rembT_h, zeros_h, out_h,
          tab_h, rtab_h, idx_d, idx_r, idx_l, idx_s, slab_v, pack_v, dom_v,
          ran_v, rel_v, row_v, sh_acc, sem0, sem1, sem2):
        cid = lax.axis_index("c")
        sid = lax.axis_index("s")
        wid = sid * NC + cid
        sems = (sem0, sem1)

        @pl.when(sid == 0)
        def _():
            pltpu.sync_copy(zeros_h, sh_acc)

        iota = lax.iota(jnp.int32, 16)

        def transpose_chunk(src_h, src_rows, dst_h, dst_base, c):
            base = jnp.minimum(c * SLAB, src_rows - SLAB)
            pltpu.sync_copy(src_h.at[:, pl.ds(base, SLAB)], slab_v)

            @pl.loop(0, SLAB)
            def _(j):
                jb = jnp.broadcast_to(j, (16,))
                for half in range(2):
                    ge = plsc.load_gather(slab_v, [2 * iota + 32 * half, jb])
                    go = plsc.load_gather(slab_v,
                                          [2 * iota + 1 + 32 * half, jb])
                    packed = plsc.pack(ge, go,
                                       format=plsc.PackFormat.INTERLEAVED)
                    pack_v[j, pl.ds(half * 16, 16)] = plsc.bitcast(
                        packed, jnp.int32)

            pltpu.sync_copy(pack_v, dst_h.at[pl.ds(dst_base + base, SLAB)])

        @pl.loop(0, (NTCHUNK + NS - 1) // NS)
        def _(ci):
            c = ci * NS + sid

            @pl.when(c < NTCHUNK)
            def _():
                transpose_chunk(tembT_h, NTYPES, tab_h, cid * TPAD, c)

        @pl.when(sid < RPAD // SLAB)
        def _():
            transpose_chunk(rembT_h, NRELS, rtab_h, cid * RPAD, sid)

        ones = jnp.ones((16,), jnp.float32)
        for b in range(2):
            @pl.loop(0, CHUNK)
            def _(t, b=b):
                row_v[b, t, pl.ds(16, 16)] = ones

        sl_w = pl.ds(wid * NCHUNK, NCHUNK)
        pltpu.sync_copy(dom_h.at[sl_w], idx_d)
        pltpu.sync_copy(ran_h.at[sl_w], idx_r)
        pltpu.sync_copy(rel_h.at[sl_w], idx_l)
        pltpu.sync_copy(seg_h.at[sl_w], idx_s)

        toff = jnp.broadcast_to(cid * TPAD, (16,)).astype(jnp.int32)
        roff = jnp.broadcast_to(cid * RPAD, (16,)).astype(jnp.int32)

        @pl.loop(0, NCHUNK)
        def _(g):
            @pl.loop(0, CHUNK, step=16)
            def _(v, g=g):
                sl = pl.ds(v, 16)
                idx_d[g, sl] = idx_d[g, sl] + toff
                idx_r[g, sl] = idx_r[g, sl] + toff
                idx_l[g, sl] = idx_l[g, sl] + roff

        plsc.subcore_barrier()

        def gather_trio(g, b):
            return (
                pltpu.make_async_copy(tab_h.at[idx_d.at[g]], dom_v.at[b],
                                      sems[b]),
                pltpu.make_async_copy(tab_h.at[idx_r.at[g]], ran_v.at[b],
                                      sems[b]),
                pltpu.make_async_copy(rtab_h.at[idx_l.at[g]], rel_v.at[b],
                                      sems[b]),
            )

        def issue(g, b):
            for cp in gather_trio(g, b):
                cp.start()

        issue(0, 0)
        issue(1, 1)

        @pl.loop(0, NCHUNK, step=2)
        def _(g0):
            for b in range(2):
                g = g0 + b
                for cp in gather_trio(g, b):
                    cp.wait()

                @pl.loop(0, CHUNK)
                def _(t, b=b):
                    s0, s1 = pl.ds(0, 16), pl.ds(16, 16)
                    d0 = plsc.bitcast(dom_v[b, t, s0], jnp.bfloat16)
                    d1 = plsc.bitcast(dom_v[b, t, s1], jnp.bfloat16)
                    r0 = plsc.bitcast(ran_v[b, t, s0], jnp.bfloat16)
                    r1 = plsc.bitcast(ran_v[b, t, s1], jnp.bfloat16)
                    l0 = plsc.bitcast(rel_v[b, t, s0], jnp.bfloat16)
                    l1 = plsc.bitcast(rel_v[b, t, s1], jnp.bfloat16)
                    e0 = d0 + l0 - r0
                    e1 = d1 + l1 - r1
                    s = e0 * e0 + e1 * e1
                    pa, pb = plsc.unpack(s, format=plsc.PackFormat.INTERLEAVED)
                    row_v[b, t, pl.ds(0, 16)] = -(pa + pb)

                pltpu.sync_copy(row_v.at[b], sh_acc.at[idx_s.at[g]], add=True)

                @pl.when(g + 2 < NCHUNK)
                def _(g=g, b=b):
                    issue(g + 2, b)

        plsc.subcore_barrier()
        rows_per_sub = NUM_SEGMENTS // NS
        pltpu.sync_copy(
            sh_acc.at[pl.ds(sid * rows_per_sub, rows_per_sub)],
            out_h.at[cid, pl.ds(sid * rows_per_sub, rows_per_sub)],
        )

    return k(dom2d, ran2d, rel2d, seg2d, tembT, rembT, zeros_hbm)[0]


def _finish(acc):
    def body(a_ref, o_ref):
        a = a_ref[0] + a_ref[1]
        sums = jnp.sum(a[:, :16], axis=1)
        cnt = a[:, 16]
        o_ref[...] = jnp.where(cnt > 0, sums / jnp.maximum(cnt, 1.0), 0.0)

    return pl.pallas_call(
        body,
        out_shape=jax.ShapeDtypeStruct((NUM_SEGMENTS,), jnp.float32),
    )(acc)


def kernel(dom_ids, ran_ids, rel_ids, segment_ids, type_emb, rel_emb):
    dom2d = dom_ids.astype(jnp.int32).reshape(T // CHUNK, CHUNK)
    ran2d = ran_ids.astype(jnp.int32).reshape(T // CHUNK, CHUNK)
    rel2d = rel_ids.astype(jnp.int32).reshape(T // CHUNK, CHUNK)
    seg2d = segment_ids.astype(jnp.int32).reshape(T // CHUNK, CHUNK)
    zeros = jnp.zeros((NUM_SEGMENTS, ACCW), jnp.float32)
    acc = _sc_kernel(dom2d, ran2d, rel2d, seg2d, type_emb.T, rel_emb.T, zeros)
    return _finish(acc)


# speedup vs baseline: 1.2175x; 1.2175x over previous
import functools

import jax
import jax.numpy as jnp
from jax import lax
from jax.experimental import pallas as pl
from jax.experimental.pallas import tpu as pltpu
from jax.experimental.pallas import tpu_sc as plsc

NUM_SEGMENTS = 4096
T = 131072
D = 64
DW = D // 2
NC = 2
NS = 16
NW = NC * NS
PER_W = T // NW
CHUNK = 128
NCHUNK = PER_W // CHUNK
ACCW = 32

NTYPES = 50000
NRELS = 1000
SLAB = 256
TPAD = ((NTYPES + SLAB - 1) // SLAB) * SLAB
RPAD = ((NRELS + SLAB - 1) // SLAB) * SLAB
NTCHUNK = TPAD // SLAB
NACHUNK = (NTCHUNK + NS - 1) // NS


def _sc_kernel(dom2d, ran2d, rel2d, seg2d, tembT, rembT, zeros_hbm):
    mesh = plsc.VectorSubcoreMesh(core_axis_name="c", subcore_axis_name="s")

    @functools.partial(
        pl.kernel,
        out_type=(
            jax.ShapeDtypeStruct((NC, NUM_SEGMENTS, ACCW), jnp.float32),
            jax.ShapeDtypeStruct((NC * TPAD, DW), jnp.int32),
            jax.ShapeDtypeStruct((NC * RPAD, DW), jnp.int32),
        ),
        mesh=mesh,
        scratch_types=[
            pltpu.VMEM((NCHUNK, CHUNK), jnp.int32),
            pltpu.VMEM((NCHUNK, CHUNK), jnp.int32),
            pltpu.VMEM((NCHUNK, CHUNK), jnp.int32),
            pltpu.VMEM((NCHUNK, CHUNK), jnp.int32),
            pltpu.VMEM((2, 64, SLAB), jnp.float32),
            pltpu.VMEM((2, SLAB, DW), jnp.int32),
            pltpu.VMEM((2, CHUNK, DW), jnp.int32),
            pltpu.VMEM((2, CHUNK, DW), jnp.int32),
            pltpu.VMEM((2, CHUNK, DW), jnp.int32),
            pltpu.VMEM((2, CHUNK, ACCW), jnp.float32),
            pltpu.VMEM_SHARED((NUM_SEGMENTS, ACCW), jnp.float32),
            pltpu.SemaphoreType.DMA,
            pltpu.SemaphoreType.DMA,
            pltpu.SemaphoreType.DMA,
            pltpu.SemaphoreType.DMA,
            pltpu.SemaphoreType.DMA,
        ],
        compiler_params=pltpu.CompilerParams(use_tc_tiling_on_sc=False,
                                             needs_layout_passes=False),
    )
    def k(dom_h, ran_h, rel_h, seg_h, tembT_h, rembT_h, zeros_h, out_h,
          tab_h, rtab_h, idx_d, idx_r, idx_l, idx_s, slab_v, pack_v, dom_v,
          ran_v, rel_v, row_v, sh_acc, sem0, sem1, semA0, semA1, semP):
        cid = lax.axis_index("c")
        sid = lax.axis_index("s")
        wid = sid * NC + cid
        sems = (sem0, sem1)

        @pl.when(sid == 0)
        def _():
            pltpu.sync_copy(zeros_h, sh_acc)

        iota = lax.iota(jnp.int32, 16)
        semsA = (semA0, semA1)

        def slab_base(c):
            return jnp.minimum(c * SLAB, NTYPES - SLAB)

        def slab_copy(ci, b):
            c = ci * NS + sid

            @pl.when(c < NTCHUNK)
            def _():
                pltpu.make_async_copy(
                    tembT_h.at[:, pl.ds(slab_base(c), SLAB)],
                    slab_v.at[b], semsA[b]).start()

        def transpose_rows(src, dst):
            @pl.loop(0, SLAB, step=4)
            def _(j0):
                for dj in range(4):
                    jb = jnp.broadcast_to(j0 + dj, (16,))
                    for half in range(2):
                        ge = plsc.load_gather(src, [2 * iota + 32 * half, jb])
                        go = plsc.load_gather(src,
                                              [2 * iota + 1 + 32 * half, jb])
                        packed = plsc.pack(ge, go,
                                           format=plsc.PackFormat.INTERLEAVED)
                        dst[j0 + dj, pl.ds(half * 16, 16)] = plsc.bitcast(
                            packed, jnp.int32)

        slab_copy(0, 0)
        slab_copy(1, 1)

        @pl.loop(0, NACHUNK + (NACHUNK % 2), step=2)
        def _(ci0):
            for b in range(2):
                ci = ci0 + b
                c = ci * NS + sid

                @pl.when(c < NTCHUNK)
                def _(ci=ci, c=c, b=b):
                    pltpu.make_async_copy(
                        tembT_h.at[:, pl.ds(slab_base(c), SLAB)],
                        slab_v.at[b], semsA[b]).wait()

                    @pl.when(ci >= 2)
                    def _():
                        pltpu.make_async_copy(
                            pack_v.at[b],
                            tab_h.at[pl.ds(0, SLAB)], semP).wait()

                    transpose_rows(slab_v.at[b], pack_v.at[b])
                    pltpu.make_async_copy(
                        pack_v.at[b],
                        tab_h.at[pl.ds(cid * TPAD + slab_base(c), SLAB)],
                        semP).start()
                    slab_copy(ci + 2, b)

        for b in range(2):
            pltpu.make_async_copy(pack_v.at[b], tab_h.at[pl.ds(0, SLAB)],
                                  semP).wait()

        @pl.when(sid >= NS - RPAD // SLAB)
        def _():
            c = sid - (NS - RPAD // SLAB)
            base = jnp.minimum(c * SLAB, NRELS - SLAB)
            pltpu.sync_copy(rembT_h.at[:, pl.ds(base, SLAB)], slab_v.at[0])
            transpose_rows(slab_v.at[0], pack_v.at[0])
            pltpu.sync_copy(pack_v.at[0],
                            rtab_h.at[pl.ds(cid * RPAD + base, SLAB)])

        ones = jnp.ones((16,), jnp.float32)
        for b in range(2):
            @pl.loop(0, CHUNK)
            def _(t, b=b):
                row_v[b, t, pl.ds(16, 16)] = ones

        sl_w = pl.ds(wid * NCHUNK, NCHUNK)
        pltpu.sync_copy(dom_h.at[sl_w], idx_d)
        pltpu.sync_copy(ran_h.at[sl_w], idx_r)
        pltpu.sync_copy(rel_h.at[sl_w], idx_l)
        pltpu.sync_copy(seg_h.at[sl_w], idx_s)

        toff = jnp.broadcast_to(cid * TPAD, (16,)).astype(jnp.int32)
        roff = jnp.broadcast_to(cid * RPAD, (16,)).astype(jnp.int32)

        @pl.loop(0, NCHUNK)
        def _(g):
            @pl.loop(0, CHUNK, step=16)
            def _(v, g=g):
                sl = pl.ds(v, 16)
                idx_d[g, sl] = idx_d[g, sl] + toff
                idx_r[g, sl] = idx_r[g, sl] + toff
                idx_l[g, sl] = idx_l[g, sl] + roff

        plsc.subcore_barrier()

        def gather_trio(g, b):
            return (
                pltpu.make_async_copy(tab_h.at[idx_d.at[g]], dom_v.at[b],
                                      sems[b]),
                pltpu.make_async_copy(tab_h.at[idx_r.at[g]], ran_v.at[b],
                                      sems[b]),
                pltpu.make_async_copy(rtab_h.at[idx_l.at[g]], rel_v.at[b],
                                      sems[b]),
            )

        def issue(g, b):
            for cp in gather_trio(g, b):
                cp.start()

        issue(0, 0)
        issue(1, 1)

        @pl.loop(0, NCHUNK, step=2)
        def _(g0):
            for b in range(2):
                g = g0 + b
                for cp in gather_trio(g, b):
                    cp.wait()

                @pl.loop(0, CHUNK)
                def _(t, b=b):
                    s0, s1 = pl.ds(0, 16), pl.ds(16, 16)
                    d0 = plsc.bitcast(dom_v[b, t, s0], jnp.bfloat16)
                    d1 = plsc.bitcast(dom_v[b, t, s1], jnp.bfloat16)
                    r0 = plsc.bitcast(ran_v[b, t, s0], jnp.bfloat16)
                    r1 = plsc.bitcast(ran_v[b, t, s1], jnp.bfloat16)
                    l0 = plsc.bitcast(rel_v[b, t, s0], jnp.bfloat16)
                    l1 = plsc.bitcast(rel_v[b, t, s1], jnp.bfloat16)
                    e0 = d0 + l0 - r0
                    e1 = d1 + l1 - r1
                    s = e0 * e0 + e1 * e1
                    pa, pb = plsc.unpack(s, format=plsc.PackFormat.INTERLEAVED)
                    row_v[b, t, pl.ds(0, 16)] = -(pa + pb)

                pltpu.sync_copy(row_v.at[b], sh_acc.at[idx_s.at[g]], add=True)

                @pl.when(g + 2 < NCHUNK)
                def _(g=g, b=b):
                    issue(g + 2, b)

        plsc.subcore_barrier()
        rows_per_sub = NUM_SEGMENTS // NS
        pltpu.sync_copy(
            sh_acc.at[pl.ds(sid * rows_per_sub, rows_per_sub)],
            out_h.at[cid, pl.ds(sid * rows_per_sub, rows_per_sub)],
        )

    return k(dom2d, ran2d, rel2d, seg2d, tembT, rembT, zeros_hbm)[0]


def _finish(acc):
    def body(a_ref, o_ref):
        a = a_ref[0] + a_ref[1]
        sums = jnp.sum(a[:, :16], axis=1)
        cnt = a[:, 16]
        o_ref[...] = jnp.where(cnt > 0, sums / jnp.maximum(cnt, 1.0), 0.0)

    return pl.pallas_call(
        body,
        out_shape=jax.ShapeDtypeStruct((NUM_SEGMENTS,), jnp.float32),
    )(acc)


def kernel(dom_ids, ran_ids, rel_ids, segment_ids, type_emb, rel_emb):
    dom2d = dom_ids.astype(jnp.int32).reshape(T // CHUNK, CHUNK)
    ran2d = ran_ids.astype(jnp.int32).reshape(T // CHUNK, CHUNK)
    rel2d = rel_ids.astype(jnp.int32).reshape(T // CHUNK, CHUNK)
    seg2d = segment_ids.astype(jnp.int32).reshape(T // CHUNK, CHUNK)
    zeros = jnp.zeros((NUM_SEGMENTS, ACCW), jnp.float32)
    acc = _sc_kernel(dom2d, ran2d, rel2d, seg2d, type_emb.T, rel_emb.T, zeros)
    return _finish(acc)


# speedup vs baseline: 2.0723x; 1.7021x over previous
import functools

import jax
import jax.numpy as jnp
from jax import lax
from jax.experimental import pallas as pl
from jax.experimental.pallas import tpu as pltpu
from jax.experimental.pallas import tpu_sc as plsc

NUM_SEGMENTS = 4096
T = 131072
D = 64
DW = D // 2
NC = 2
NS = 16
NW = NC * NS
PER_W = T // NW
CHUNK = 128
NCHUNK = PER_W // CHUNK
ACCW = 32

NTYPES = 50000
NRELS = 1000
SLAB = 256
TPAD = ((NTYPES + SLAB - 1) // SLAB) * SLAB
RPAD = ((NRELS + SLAB - 1) // SLAB) * SLAB
NTCHUNK = TPAD // SLAB
NACHUNK = (NTCHUNK + NS - 1) // NS


def _sc_kernel(dom2d, ran2d, rel2d, seg2d, tembT, rembT, zeros_hbm):
    mesh = plsc.VectorSubcoreMesh(core_axis_name="c", subcore_axis_name="s")

    @functools.partial(
        pl.kernel,
        out_type=(
            jax.ShapeDtypeStruct((NC, NUM_SEGMENTS, ACCW), jnp.float32),
            jax.ShapeDtypeStruct((NC * TPAD, DW), jnp.int32),
            jax.ShapeDtypeStruct((NC * RPAD, DW), jnp.int32),
        ),
        mesh=mesh,
        scratch_types=[
            pltpu.VMEM((NCHUNK, CHUNK), jnp.int32),
            pltpu.VMEM((NCHUNK, CHUNK), jnp.int32),
            pltpu.VMEM((NCHUNK, CHUNK), jnp.int32),
            pltpu.VMEM((NCHUNK, CHUNK), jnp.int32),
            pltpu.VMEM((2, SLAB, 64), jnp.float32),
            pltpu.VMEM((2, SLAB, DW), jnp.int32),
            pltpu.VMEM((2, CHUNK, DW), jnp.int32),
            pltpu.VMEM((2, CHUNK, DW), jnp.int32),
            pltpu.VMEM((2, CHUNK, DW), jnp.int32),
            pltpu.VMEM((2, CHUNK, ACCW), jnp.float32),
            pltpu.VMEM_SHARED((NUM_SEGMENTS, ACCW), jnp.float32),
            pltpu.SemaphoreType.DMA,
            pltpu.SemaphoreType.DMA,
            pltpu.SemaphoreType.DMA,
            pltpu.SemaphoreType.DMA,
            pltpu.SemaphoreType.DMA,
        ],
        compiler_params=pltpu.CompilerParams(use_tc_tiling_on_sc=False,
                                             needs_layout_passes=False),
    )
    def k(dom_h, ran_h, rel_h, seg_h, tembT_h, rembT_h, zeros_h, out_h,
          tab_h, rtab_h, idx_d, idx_r, idx_l, idx_s, slab_v, pack_v, dom_v,
          ran_v, rel_v, row_v, sh_acc, sem0, sem1, semA0, semA1, semP):
        cid = lax.axis_index("c")
        sid = lax.axis_index("s")
        wid = sid * NC + cid
        sems = (sem0, sem1)

        @pl.when(sid == 0)
        def _():
            pltpu.sync_copy(zeros_h, sh_acc)

        semsA = (semA0, semA1)

        def slab_base(c):
            return jnp.minimum(c * SLAB, NTYPES - SLAB)

        def slab_copy(ci, b):
            c = ci * NS + sid

            @pl.when(c < NTCHUNK)
            def _():
                pltpu.make_async_copy(
                    tembT_h.at[pl.ds(slab_base(c), SLAB)],
                    slab_v.at[b], semsA[b]).start()

        def transpose_rows(src, dst):
            @pl.loop(0, SLAB, step=4)
            def _(j0):
                for dj in range(4):
                    j = j0 + dj
                    for half in range(2):
                        a = src[j, pl.ds(32 * half, 16)]
                        c2 = src[j, pl.ds(32 * half + 16, 16)]
                        packed = plsc.pack(a, c2,
                                           format=plsc.PackFormat.INTERLEAVED)
                        dst[j, pl.ds(half * 16, 16)] = plsc.bitcast(
                            packed, jnp.int32)

        slab_copy(0, 0)
        slab_copy(1, 1)

        @pl.loop(0, NACHUNK + (NACHUNK % 2), step=2)
        def _(ci0):
            for b in range(2):
                ci = ci0 + b
                c = ci * NS + sid

                @pl.when(c < NTCHUNK)
                def _(ci=ci, c=c, b=b):
                    pltpu.make_async_copy(
                        tembT_h.at[pl.ds(slab_base(c), SLAB)],
                        slab_v.at[b], semsA[b]).wait()

                    @pl.when(ci >= 2)
                    def _():
                        pltpu.make_async_copy(
                            pack_v.at[b],
                            tab_h.at[pl.ds(0, SLAB)], semP).wait()

                    transpose_rows(slab_v.at[b], pack_v.at[b])
                    pltpu.make_async_copy(
                        pack_v.at[b],
                        tab_h.at[pl.ds(cid * TPAD + slab_base(c), SLAB)],
                        semP).start()
                    slab_copy(ci + 2, b)

        for b in range(2):
            pltpu.make_async_copy(pack_v.at[b], tab_h.at[pl.ds(0, SLAB)],
                                  semP).wait()

        @pl.when(sid >= NS - RPAD // SLAB)
        def _():
            c = sid - (NS - RPAD // SLAB)
            base = jnp.minimum(c * SLAB, NRELS - SLAB)
            pltpu.sync_copy(rembT_h.at[pl.ds(base, SLAB)], slab_v.at[0])
            transpose_rows(slab_v.at[0], pack_v.at[0])
            pltpu.sync_copy(pack_v.at[0],
                            rtab_h.at[pl.ds(cid * RPAD + base, SLAB)])

        ones = jnp.ones((16,), jnp.float32)
        for b in range(2):
            @pl.loop(0, CHUNK)
            def _(t, b=b):
                row_v[b, t, pl.ds(16, 16)] = ones

        sl_w = pl.ds(wid * NCHUNK, NCHUNK)
        pltpu.sync_copy(dom_h.at[sl_w], idx_d)
        pltpu.sync_copy(ran_h.at[sl_w], idx_r)
        pltpu.sync_copy(rel_h.at[sl_w], idx_l)
        pltpu.sync_copy(seg_h.at[sl_w], idx_s)

        toff = jnp.broadcast_to(cid * TPAD, (16,)).astype(jnp.int32)
        roff = jnp.broadcast_to(cid * RPAD, (16,)).astype(jnp.int32)

        @pl.loop(0, NCHUNK)
        def _(g):
            @pl.loop(0, CHUNK, step=16)
            def _(v, g=g):
                sl = pl.ds(v, 16)
                idx_d[g, sl] = idx_d[g, sl] + toff
                idx_r[g, sl] = idx_r[g, sl] + toff
                idx_l[g, sl] = idx_l[g, sl] + roff

        plsc.subcore_barrier()

        def gather_trio(g, b):
            return (
                pltpu.make_async_copy(tab_h.at[idx_d.at[g]], dom_v.at[b],
                                      sems[b]),
                pltpu.make_async_copy(tab_h.at[idx_r.at[g]], ran_v.at[b],
                                      sems[b]),
                pltpu.make_async_copy(rtab_h.at[idx_l.at[g]], rel_v.at[b],
                                      sems[b]),
            )

        def issue(g, b):
            for cp in gather_trio(g, b):
                cp.start()

        issue(0, 0)
        issue(1, 1)

        @pl.loop(0, NCHUNK, step=2)
        def _(g0):
            for b in range(2):
                g = g0 + b
                for cp in gather_trio(g, b):
                    cp.wait()

                @pl.loop(0, CHUNK)
                def _(t, b=b):
                    s0, s1 = pl.ds(0, 16), pl.ds(16, 16)
                    d0 = plsc.bitcast(dom_v[b, t, s0], jnp.bfloat16)
                    d1 = plsc.bitcast(dom_v[b, t, s1], jnp.bfloat16)
                    r0 = plsc.bitcast(ran_v[b, t, s0], jnp.bfloat16)
                    r1 = plsc.bitcast(ran_v[b, t, s1], jnp.bfloat16)
                    l0 = plsc.bitcast(rel_v[b, t, s0], jnp.bfloat16)
                    l1 = plsc.bitcast(rel_v[b, t, s1], jnp.bfloat16)
                    e0 = d0 + l0 - r0
                    e1 = d1 + l1 - r1
                    s = e0 * e0 + e1 * e1
                    pa, pb = plsc.unpack(s, format=plsc.PackFormat.INTERLEAVED)
                    row_v[b, t, pl.ds(0, 16)] = -(pa + pb)

                pltpu.sync_copy(row_v.at[b], sh_acc.at[idx_s.at[g]], add=True)

                @pl.when(g + 2 < NCHUNK)
                def _(g=g, b=b):
                    issue(g + 2, b)

        plsc.subcore_barrier()
        rows_per_sub = NUM_SEGMENTS // NS
        pltpu.sync_copy(
            sh_acc.at[pl.ds(sid * rows_per_sub, rows_per_sub)],
            out_h.at[cid, pl.ds(sid * rows_per_sub, rows_per_sub)],
        )

    return k(dom2d, ran2d, rel2d, seg2d, tembT, rembT, zeros_hbm)[0]


def _finish(acc):
    def body(a_ref, o_ref):
        a = a_ref[0] + a_ref[1]
        sums = jnp.sum(a[:, :16], axis=1)
        cnt = a[:, 16]
        o_ref[...] = jnp.where(cnt > 0, sums / jnp.maximum(cnt, 1.0), 0.0)

    return pl.pallas_call(
        body,
        out_shape=jax.ShapeDtypeStruct((NUM_SEGMENTS,), jnp.float32),
    )(acc)


def kernel(dom_ids, ran_ids, rel_ids, segment_ids, type_emb, rel_emb):
    dom2d = dom_ids.astype(jnp.int32).reshape(T // CHUNK, CHUNK)
    ran2d = ran_ids.astype(jnp.int32).reshape(T // CHUNK, CHUNK)
    rel2d = rel_ids.astype(jnp.int32).reshape(T // CHUNK, CHUNK)
    seg2d = segment_ids.astype(jnp.int32).reshape(T // CHUNK, CHUNK)
    zeros = jnp.zeros((NUM_SEGMENTS, ACCW), jnp.float32)
    acc = _sc_kernel(dom2d, ran2d, rel2d, seg2d, type_emb, rel_emb, zeros)
    return _finish(acc)


# speedup vs baseline: 2.1293x; 1.0275x over previous
import functools

import jax
import jax.numpy as jnp
from jax import lax
from jax.experimental import pallas as pl
from jax.experimental.pallas import tpu as pltpu
from jax.experimental.pallas import tpu_sc as plsc

NUM_SEGMENTS = 4096
T = 131072
D = 64
DW = D // 2
NC = 2
NS = 16
NW = NC * NS
PER_W = T // NW
CHUNK = 128
NCHUNK = PER_W // CHUNK
ACCW = 32

NTYPES = 50000
NRELS = 1000
SLAB = 256
TPAD = ((NTYPES + SLAB - 1) // SLAB) * SLAB
RPAD = ((NRELS + SLAB - 1) // SLAB) * SLAB
NTCHUNK = TPAD // SLAB
NACHUNK = (NTCHUNK + NS - 1) // NS


def _sc_kernel(dom2d, ran2d, rel2d, seg2d, tembT, rembT, zeros_hbm):
    mesh = plsc.VectorSubcoreMesh(core_axis_name="c", subcore_axis_name="s")

    @functools.partial(
        pl.kernel,
        out_type=(
            jax.ShapeDtypeStruct((NC, NUM_SEGMENTS, ACCW), jnp.float32),
            jax.ShapeDtypeStruct((NC * TPAD, DW), jnp.int32),
            jax.ShapeDtypeStruct((NC * RPAD, DW), jnp.int32),
        ),
        mesh=mesh,
        scratch_types=[
            pltpu.VMEM((NCHUNK, CHUNK), jnp.int32),
            pltpu.VMEM((NCHUNK, CHUNK), jnp.int32),
            pltpu.VMEM((NCHUNK, CHUNK), jnp.int32),
            pltpu.VMEM((NCHUNK, CHUNK), jnp.int32),
            pltpu.VMEM((2, SLAB, 64), jnp.float32),
            pltpu.VMEM((2, SLAB, DW), jnp.int32),
            pltpu.VMEM((2, CHUNK, DW), jnp.int32),
            pltpu.VMEM((2, CHUNK, DW), jnp.int32),
            pltpu.VMEM((2, CHUNK, DW), jnp.int32),
            pltpu.VMEM((2, CHUNK, ACCW), jnp.float32),
            pltpu.VMEM_SHARED((NUM_SEGMENTS, ACCW), jnp.float32),
            pltpu.SemaphoreType.DMA,
            pltpu.SemaphoreType.DMA,
            pltpu.SemaphoreType.DMA,
            pltpu.SemaphoreType.DMA,
            pltpu.SemaphoreType.DMA,
        ],
        compiler_params=pltpu.CompilerParams(use_tc_tiling_on_sc=False,
                                             needs_layout_passes=False),
    )
    def k(dom_h, ran_h, rel_h, seg_h, tembT_h, rembT_h, zeros_h, out_h,
          tab_h, rtab_h, idx_d, idx_r, idx_l, idx_s, slab_v, pack_v, dom_v,
          ran_v, rel_v, row_v, sh_acc, sem0, sem1, semA0, semA1, semP):
        cid = lax.axis_index("c")
        sid = lax.axis_index("s")
        wid = sid * NC + cid
        sems = (sem0, sem1)

        @pl.when(sid == 0)
        def _():
            pltpu.sync_copy(zeros_h, sh_acc)

        semsA = (semA0, semA1)

        def slab_base(c):
            return jnp.minimum(c * SLAB, NTYPES - SLAB)

        def slab_copy(ci, b):
            c = ci * NS + sid

            @pl.when(c < NTCHUNK)
            def _():
                pltpu.make_async_copy(
                    tembT_h.at[pl.ds(slab_base(c), SLAB)],
                    slab_v.at[b], semsA[b]).start()

        def transpose_rows(src, dst):
            @pl.loop(0, SLAB, step=8)
            def _(j0):
                for dj in range(8):
                    j = j0 + dj
                    for half in range(2):
                        a = src[j, pl.ds(32 * half, 16)]
                        c2 = src[j, pl.ds(32 * half + 16, 16)]
                        packed = plsc.pack(a, c2,
                                           format=plsc.PackFormat.INTERLEAVED)
                        dst[j, pl.ds(half * 16, 16)] = plsc.bitcast(
                            packed, jnp.int32)

        slab_copy(0, 0)
        slab_copy(1, 1)

        @pl.loop(0, NACHUNK + (NACHUNK % 2), step=2)
        def _(ci0):
            for b in range(2):
                ci = ci0 + b
                c = ci * NS + sid

                @pl.when(c < NTCHUNK)
                def _(ci=ci, c=c, b=b):
                    pltpu.make_async_copy(
                        tembT_h.at[pl.ds(slab_base(c), SLAB)],
                        slab_v.at[b], semsA[b]).wait()

                    @pl.when(ci >= 2)
                    def _():
                        pltpu.make_async_copy(
                            pack_v.at[b],
                            tab_h.at[pl.ds(0, SLAB)], semP).wait()

                    transpose_rows(slab_v.at[b], pack_v.at[b])
                    pltpu.make_async_copy(
                        pack_v.at[b],
                        tab_h.at[pl.ds(cid * TPAD + slab_base(c), SLAB)],
                        semP).start()
                    slab_copy(ci + 2, b)

        for b in range(2):
            pltpu.make_async_copy(pack_v.at[b], tab_h.at[pl.ds(0, SLAB)],
                                  semP).wait()

        @pl.when(sid >= NS - RPAD // SLAB)
        def _():
            c = sid - (NS - RPAD // SLAB)
            base = jnp.minimum(c * SLAB, NRELS - SLAB)
            pltpu.sync_copy(rembT_h.at[pl.ds(base, SLAB)], slab_v.at[0])
            transpose_rows(slab_v.at[0], pack_v.at[0])
            pltpu.sync_copy(pack_v.at[0],
                            rtab_h.at[pl.ds(cid * RPAD + base, SLAB)])

        ones = jnp.ones((16,), jnp.float32)
        for b in range(2):
            @pl.loop(0, CHUNK)
            def _(t, b=b):
                row_v[b, t, pl.ds(16, 16)] = ones

        sl_w = pl.ds(wid * NCHUNK, NCHUNK)
        pltpu.sync_copy(dom_h.at[sl_w], idx_d)
        pltpu.sync_copy(ran_h.at[sl_w], idx_r)
        pltpu.sync_copy(rel_h.at[sl_w], idx_l)
        pltpu.sync_copy(seg_h.at[sl_w], idx_s)

        toff = jnp.broadcast_to(cid * TPAD, (16,)).astype(jnp.int32)
        roff = jnp.broadcast_to(cid * RPAD, (16,)).astype(jnp.int32)

        @pl.loop(0, NCHUNK)
        def _(g):
            @pl.loop(0, CHUNK, step=16)
            def _(v, g=g):
                sl = pl.ds(v, 16)
                idx_d[g, sl] = idx_d[g, sl] + toff
                idx_r[g, sl] = idx_r[g, sl] + toff
                idx_l[g, sl] = idx_l[g, sl] + roff

        plsc.subcore_barrier()

        def gather_trio(g, b):
            return (
                pltpu.make_async_copy(tab_h.at[idx_d.at[g]], dom_v.at[b],
                                      sems[b]),
                pltpu.make_async_copy(tab_h.at[idx_r.at[g]], ran_v.at[b],
                                      sems[b]),
                pltpu.make_async_copy(rtab_h.at[idx_l.at[g]], rel_v.at[b],
                                      sems[b]),
            )

        def issue(g, b):
            for cp in gather_trio(g, b):
                cp.start()

        issue(0, 0)
        issue(1, 1)

        @pl.loop(0, NCHUNK, step=2)
        def _(g0):
            for b in range(2):
                g = g0 + b
                for cp in gather_trio(g, b):
                    cp.wait()

                @pl.loop(0, CHUNK)
                def _(t, b=b):
                    s0, s1 = pl.ds(0, 16), pl.ds(16, 16)
                    d0 = plsc.bitcast(dom_v[b, t, s0], jnp.bfloat16)
                    d1 = plsc.bitcast(dom_v[b, t, s1], jnp.bfloat16)
                    r0 = plsc.bitcast(ran_v[b, t, s0], jnp.bfloat16)
                    r1 = plsc.bitcast(ran_v[b, t, s1], jnp.bfloat16)
                    l0 = plsc.bitcast(rel_v[b, t, s0], jnp.bfloat16)
                    l1 = plsc.bitcast(rel_v[b, t, s1], jnp.bfloat16)
                    e0 = d0 + l0 - r0
                    e1 = d1 + l1 - r1
                    s = e0 * e0 + e1 * e1
                    pa, pb = plsc.unpack(s, format=plsc.PackFormat.INTERLEAVED)
                    row_v[b, t, pl.ds(0, 16)] = -(pa + pb)

                pltpu.sync_copy(row_v.at[b], sh_acc.at[idx_s.at[g]], add=True)

                @pl.when(g + 2 < NCHUNK)
                def _(g=g, b=b):
                    issue(g + 2, b)

        plsc.subcore_barrier()
        rows_per_sub = NUM_SEGMENTS // NS
        pltpu.sync_copy(
            sh_acc.at[pl.ds(sid * rows_per_sub, rows_per_sub)],
            out_h.at[cid, pl.ds(sid * rows_per_sub, rows_per_sub)],
        )

    return k(dom2d, ran2d, rel2d, seg2d, tembT, rembT, zeros_hbm)[0]


def _finish(acc):
    half = NUM_SEGMENTS // 4

    def body(a_ref, o_ref):
        s = a_ref[0:half, :] + a_ref[half:2 * half, :]
        cols = []
        for k in range(4):
            sums = jnp.sum(s[:, 32 * k:32 * k + 16], axis=1, keepdims=True)
            cnt = s[:, 32 * k + 16:32 * k + 17]
            cols.append(jnp.where(cnt > 0, sums / jnp.maximum(cnt, 1.0), 0.0))
        o_ref[...] = jnp.concatenate(cols, axis=1)

    out = pl.pallas_call(
        body,
        out_shape=jax.ShapeDtypeStruct((half, 4), jnp.float32),
    )(acc.reshape(2 * half, 128))
    return out.reshape(NUM_SEGMENTS)


def kernel(dom_ids, ran_ids, rel_ids, segment_ids, type_emb, rel_emb):
    dom2d = dom_ids.astype(jnp.int32).reshape(T // CHUNK, CHUNK)
    ran2d = ran_ids.astype(jnp.int32).reshape(T // CHUNK, CHUNK)
    rel2d = rel_ids.astype(jnp.int32).reshape(T // CHUNK, CHUNK)
    seg2d = segment_ids.astype(jnp.int32).reshape(T // CHUNK, CHUNK)
    zeros = jnp.zeros((NUM_SEGMENTS, ACCW), jnp.float32)
    acc = _sc_kernel(dom2d, ran2d, rel2d, seg2d, type_emb, rel_emb, zeros)
    return _finish(acc)


# speedup vs baseline: 2.3006x; 1.0804x over previous
import functools

import jax
import jax.numpy as jnp
from jax import lax
from jax.experimental import pallas as pl
from jax.experimental.pallas import tpu as pltpu
from jax.experimental.pallas import tpu_sc as plsc

NUM_SEGMENTS = 4096
T = 131072
D = 64
NC = 2
NS = 16
NW = NC * NS
PER_W = T // NW
CHUNK = 128
NCHUNK = PER_W // CHUNK
ACCW = 32


def _sc_kernel(dom2d, ran2d, rel2d, seg2d, temb, remb, zeros_hbm):
    mesh = plsc.VectorSubcoreMesh(core_axis_name="c", subcore_axis_name="s")

    @functools.partial(
        pl.kernel,
        out_type=jax.ShapeDtypeStruct((NC, NUM_SEGMENTS, ACCW), jnp.float32),
        mesh=mesh,
        scratch_types=[
            pltpu.VMEM((NCHUNK, CHUNK), jnp.int32),
            pltpu.VMEM((NCHUNK, CHUNK), jnp.int32),
            pltpu.VMEM((NCHUNK, CHUNK), jnp.int32),
            pltpu.VMEM((NCHUNK, CHUNK), jnp.int32),
            pltpu.VMEM((2, CHUNK, D), jnp.bfloat16),
            pltpu.VMEM((2, CHUNK, D), jnp.bfloat16),
            pltpu.VMEM((2, CHUNK, D), jnp.bfloat16),
            pltpu.VMEM((2, CHUNK, ACCW), jnp.float32),
            pltpu.VMEM_SHARED((NUM_SEGMENTS, ACCW), jnp.float32),
            pltpu.SemaphoreType.DMA,
            pltpu.SemaphoreType.DMA,
        ],
        compiler_params=pltpu.CompilerParams(use_tc_tiling_on_sc=False,
                                             needs_layout_passes=False),
    )
    def k(dom_h, ran_h, rel_h, seg_h, temb_h, remb_h, zeros_h, out_h,
          idx_d, idx_r, idx_l, idx_s, dom_v, ran_v, rel_v, row_v,
          shared_acc, sem0, sem1):
        cid = lax.axis_index("c")
        sid = lax.axis_index("s")
        wid = sid * NC + cid
        sems = (sem0, sem1)

        @pl.when(sid == 0)
        def _():
            pltpu.sync_copy(zeros_h, shared_acc)

        ones = jnp.ones((16,), jnp.float32)
        for b in range(2):
            @pl.loop(0, CHUNK)
            def _(t, b=b):
                row_v[b, t, pl.ds(16, 16)] = ones

        sl_w = pl.ds(wid * NCHUNK, NCHUNK)
        pltpu.sync_copy(dom_h.at[sl_w], idx_d)
        pltpu.sync_copy(ran_h.at[sl_w], idx_r)
        pltpu.sync_copy(rel_h.at[sl_w], idx_l)
        pltpu.sync_copy(seg_h.at[sl_w], idx_s)

        plsc.subcore_barrier()

        def gather_trio(g, b):
            return (
                pltpu.make_async_copy(temb_h.at[idx_d.at[g]], dom_v.at[b], sems[b]),
                pltpu.make_async_copy(temb_h.at[idx_r.at[g]], ran_v.at[b], sems[b]),
                pltpu.make_async_copy(remb_h.at[idx_l.at[g]], rel_v.at[b], sems[b]),
            )

        def issue(g, b):
            for cp in gather_trio(g, b):
                cp.start()

        issue(0, 0)
        issue(1, 1)

        @pl.loop(0, NCHUNK, step=2)
        def _(g0):
            for b in range(2):
                g = g0 + b
                for cp in gather_trio(g, b):
                    cp.wait()

                @pl.loop(0, CHUNK)
                def _(t, b=b):
                    s0, s1 = pl.ds(0, 32), pl.ds(32, 32)
                    e0 = dom_v[b, t, s0] + rel_v[b, t, s0] - ran_v[b, t, s0]
                    e1 = dom_v[b, t, s1] + rel_v[b, t, s1] - ran_v[b, t, s1]
                    s = e0 * e0 + e1 * e1
                    pa, pb = plsc.unpack(s, format=plsc.PackFormat.INTERLEAVED)
                    row_v[b, t, pl.ds(0, 16)] = -(pa + pb)

                pltpu.sync_copy(row_v.at[b], shared_acc.at[idx_s.at[g]],
                                add=True)

                @pl.when(g + 2 < NCHUNK)
                def _(g=g, b=b):
                    issue(g + 2, b)

        plsc.subcore_barrier()
        rows_per_sub = NUM_SEGMENTS // NS
        pltpu.sync_copy(
            shared_acc.at[pl.ds(sid * rows_per_sub, rows_per_sub)],
            out_h.at[cid, pl.ds(sid * rows_per_sub, rows_per_sub)],
        )

    return k(dom2d, ran2d, rel2d, seg2d, temb, remb, zeros_hbm)


def _finish(acc):
    half = NUM_SEGMENTS // 4

    def body(a_ref, o_ref):
        s = a_ref[0:half, :] + a_ref[half:2 * half, :]
        cols = []
        for k in range(4):
            sums = jnp.sum(s[:, 32 * k:32 * k + 16], axis=1, keepdims=True)
            cnt = s[:, 32 * k + 16:32 * k + 17]
            cols.append(jnp.where(cnt > 0, sums / jnp.maximum(cnt, 1.0), 0.0))
        o_ref[...] = jnp.concatenate(cols, axis=1)

    out = pl.pallas_call(
        body,
        out_shape=jax.ShapeDtypeStruct((half, 4), jnp.float32),
    )(acc.reshape(2 * half, 128))
    return out.reshape(NUM_SEGMENTS)


def kernel(dom_ids, ran_ids, rel_ids, segment_ids, type_emb, rel_emb):
    dom2d = dom_ids.astype(jnp.int32).reshape(T // CHUNK, CHUNK)
    ran2d = ran_ids.astype(jnp.int32).reshape(T // CHUNK, CHUNK)
    rel2d = rel_ids.astype(jnp.int32).reshape(T // CHUNK, CHUNK)
    seg2d = segment_ids.astype(jnp.int32).reshape(T // CHUNK, CHUNK)
    zeros = jnp.zeros((NUM_SEGMENTS, ACCW), jnp.float32)
    acc = _sc_kernel(dom2d, ran2d, rel2d, seg2d,
                     type_emb.astype(jnp.bfloat16),
                     rel_emb.astype(jnp.bfloat16), zeros)
    return _finish(acc)
